# baseline (speedup 1.0000x reference)
import functools

import jax
import jax.numpy as jnp
import numpy as np
from jax import lax
from jax.experimental import pallas as pl
from jax.experimental.pallas import tpu as pltpu, tpu_sc as plsc

B = 4
H = 224
W = 224
VOX = 64
RPC = 8
NUM_CHUNKS = H // RPC
VREGS_PER_ROW = W // 16
GROUP = 7
ZH = 32
SLAB = 8 * VOX * ZH
ONE_MINUS = float(1.0 - 2.0 ** -24)


def _voxel_body(rgb_hbm, depth_hbm, um_hbm, out_hbm,
                slab_r, slab_g, slab_b,
                da, db_, ra, rb, ga, gb, ba, bb_, ubuf,
                stg0, stg1, stg2, stg3,
                sem0, sem1, sem2, sem3):
    wid = lax.axis_index("c") * 16 + lax.axis_index("s")
    bb = wid // 8
    ss = wid % 8

    pltpu.sync_copy(um_hbm, ubuf)

    zeros16 = jnp.zeros((16,), jnp.float32)

    def _zero(i, carry):
        slab_r[pl.ds(i * 16, 16)] = zeros16
        slab_g[pl.ds(i * 16, 16)] = zeros16
        slab_b[pl.ds(i * 16, 16)] = zeros16
        return carry

    def _in_copies(ck, slot, sem):
        r0 = ck * RPC
        cps = []
        for plane, bufa, bufb in (
            (depth_hbm.at[bb, 0], da, db_),
            (rgb_hbm.at[bb, 0], ra, rb),
            (rgb_hbm.at[bb, 1], ga, gb),
            (rgb_hbm.at[bb, 2], ba, bb_),
        ):
            cps.append((plane.at[pl.ds(r0, RPC), pl.ds(0, 128)],
                        bufa.at[slot], sem))
            cps.append((plane.at[pl.ds(r0, RPC), pl.ds(128, 96)],
                        bufb.at[slot], sem))
        return cps

    def _start(ck, slot, sem):
        for src, dst, s in _in_copies(ck, slot, sem):
            pltpu.async_copy(src, dst, s)

    def _wait(ck, slot, sem):
        for src, dst, s in _in_copies(ck, slot, sem):
            pltpu.make_async_copy(src, dst, s).wait()

    def _compute_chunk(ck, slot):
        ums = [ubuf[pl.ds(j * 16, 16)] for j in range(VREGS_PER_ROW)]

        def _row(r, carry2):
            row = ck * RPC + r
            vminus = row.astype(jnp.float32) - 112.0
            for g in range(VREGS_PER_ROW // GROUP):
                xys, wins, sls = [], [], []
                for j in range(g * GROUP, (g + 1) * GROUP):
                    if j < 8:
                        sl = (slot, r, pl.ds(16 * j, 16))
                        dv = da[sl]
                    else:
                        sl = (slot, r, pl.ds(16 * j - 128, 16))
                        dv = db_[sl]
                    xf = ums[j] * dv / 112.0
                    yf = vminus * dv / 112.0
                    ix = ((xf + 1.0) * 32.0).astype(jnp.int32)
                    iy = ((yf + 1.0) * 32.0).astype(jnp.int32)
                    iz = ((dv + 1.0) * 32.0).astype(jnp.int32)
                    ok = (dv > 0.1) & (dv < ONE_MINUS) & (iy < 64)
                    ok &= (ix >> 3) == ss
                    lidx = ((ix & 7) << 11) | (iy << 5) | (iz & 31)
                    _, win = plsc.scan_count(lidx, mask=ok)
                    xys.append(lidx)
                    wins.append(win)
                    sls.append((sl, j < 8))
                for t in range(GROUP):
                    sl, lo = sls[t]
                    plsc.store_scatter(slab_r, [xys[t]],
                                       ra[sl] if lo else rb[sl],
                                       mask=wins[t])
                    plsc.store_scatter(slab_g, [xys[t]],
                                       ga[sl] if lo else gb[sl],
                                       mask=wins[t])
                    plsc.store_scatter(slab_b, [xys[t]],
                                       ba[sl] if lo else bb_[sl],
                                       mask=wins[t])
            return carry2

        lax.fori_loop(0, RPC, _row, 0)

    _start(0, 0, sem0)
    lax.fori_loop(0, SLAB // 16, _zero, 0, unroll=4)

    def _pair(k, carry):
        ck0 = 2 * k
        _start(ck0 + 1, 1, sem1)
        _wait(ck0, 0, sem0)
        _compute_chunk(ck0, 0)

        @pl.when(ck0 + 2 < NUM_CHUNKS)
        def _():
            _start(ck0 + 2, 0, sem0)

        _wait(ck0 + 1, 1, sem1)
        _compute_chunk(ck0 + 1, 1)
        return carry

    lax.fori_loop(0, NUM_CHUNKS // 2, _pair, 0)

    zeros_row = jnp.zeros((16,), jnp.float32)

    def _zstage(y, carry):
        for stg in (stg0, stg1, stg2, stg3):
            stg[y, pl.ds(0, 16)] = zeros_row
            stg[y, pl.ds(16, 16)] = zeros_row
        return carry

    lax.fori_loop(0, VOX, _zstage, 0, unroll=4)

    planes = [(c, xl) for c in range(3) for xl in range(8)]
    slabs = (slab_r, slab_g, slab_b)
    stgs = (stg0, stg1, stg2, stg3)
    sems = (sem0, sem1, sem2, sem3)
    NST = 4

    def _repack(slab, stg, xl):
        def _rrow(y, carry):
            base = (xl * VOX + y) * ZH
            stg[y, pl.ds(32, 16)] = slab[pl.ds(base, 16)]
            stg[y, pl.ds(48, 16)] = slab[pl.ds(base + 16, 16)]
            return carry

        lax.fori_loop(0, VOX, _rrow, 0, unroll=4)

    for i, (c, xl) in enumerate(planes):
        stg = stgs[i % NST]
        sem = sems[i % NST]
        if i >= NST:
            pc, pxl = planes[i - NST]
            pltpu.make_async_copy(
                stg,
                out_hbm.at[bb, pc, 8 * ss + pxl], sem).wait()
        _repack(slabs[c], stg, xl)
        pltpu.async_copy(stg,
                         out_hbm.at[bb, c, 8 * ss + xl], sem)
    for i in range(len(planes) - NST, len(planes)):
        c, xl = planes[i]
        pltpu.make_async_copy(stgs[i % NST],
                              out_hbm.at[bb, c, 8 * ss + xl],
                              sems[i % NST]).wait()


@jax.jit
def kernel(rgb, depth):
    um = jnp.asarray(np.arange(W, dtype=np.float32) - 112.0)
    mesh = plsc.VectorSubcoreMesh(core_axis_name="c", subcore_axis_name="s")
    call = functools.partial(
        pl.kernel,
        mesh=mesh,
        compiler_params=pltpu.CompilerParams(needs_layout_passes=False),
        out_type=jax.ShapeDtypeStruct((B, 3, VOX, VOX, VOX), jnp.float32),
        scratch_types=[
            pltpu.VMEM((SLAB,), jnp.float32),
            pltpu.VMEM((SLAB,), jnp.float32),
            pltpu.VMEM((SLAB,), jnp.float32),
            pltpu.VMEM((2, RPC, 128), jnp.float32),
            pltpu.VMEM((2, RPC, 96), jnp.float32),
            pltpu.VMEM((2, RPC, 128), jnp.float32),
            pltpu.VMEM((2, RPC, 96), jnp.float32),
            pltpu.VMEM((2, RPC, 128), jnp.float32),
            pltpu.VMEM((2, RPC, 96), jnp.float32),
            pltpu.VMEM((2, RPC, 128), jnp.float32),
            pltpu.VMEM((2, RPC, 96), jnp.float32),
            pltpu.VMEM((W,), jnp.float32),
            pltpu.VMEM((VOX, VOX), jnp.float32),
            pltpu.VMEM((VOX, VOX), jnp.float32),
            pltpu.VMEM((VOX, VOX), jnp.float32),
            pltpu.VMEM((VOX, VOX), jnp.float32),
            pltpu.SemaphoreType.DMA,
            pltpu.SemaphoreType.DMA,
            pltpu.SemaphoreType.DMA,
            pltpu.SemaphoreType.DMA,
        ],
    )(_voxel_body)
    return call(rgb, depth, um)


# speedup vs baseline: 1.2771x; 1.2771x over previous
import functools

import jax
import jax.numpy as jnp
import numpy as np
from jax import lax
from jax.experimental import pallas as pl
from jax.experimental.pallas import tpu as pltpu, tpu_sc as plsc

B = 4
H = 224
W = 224
VOX = 64
RPC = 16
NUM_CHUNKS = H // RPC
VREGS_PER_ROW = W // 16
GROUP = 7
ZH = 32
SLAB = 8 * VOX * ZH
ONE_MINUS = float(1.0 - 2.0 ** -24)


def _voxel_body(rgb_hbm, depth_hbm, um_hbm, out_hbm,
                slab_r, slab_g, slab_b,
                da, db_, ra, rb, ga, gb, ba, bb_, ubuf,
                stg0, stg1, stg2, stg3,
                sem0, sem1, sem2, sem3):
    wid = lax.axis_index("c") * 16 + lax.axis_index("s")
    bb = wid // 8
    ss = wid % 8

    pltpu.sync_copy(um_hbm, ubuf)

    zeros16 = jnp.zeros((16,), jnp.float32)

    def _zero(i, carry):
        slab_r[pl.ds(i * 16, 16)] = zeros16
        slab_g[pl.ds(i * 16, 16)] = zeros16
        slab_b[pl.ds(i * 16, 16)] = zeros16
        return carry

    def _in_copies(ck, slot, sem):
        r0 = ck * RPC
        cps = []
        for plane, bufa, bufb in (
            (depth_hbm.at[bb, 0], da, db_),
            (rgb_hbm.at[bb, 0], ra, rb),
            (rgb_hbm.at[bb, 1], ga, gb),
            (rgb_hbm.at[bb, 2], ba, bb_),
        ):
            cps.append((plane.at[pl.ds(r0, RPC), pl.ds(0, 128)],
                        bufa.at[slot], sem))
            cps.append((plane.at[pl.ds(r0, RPC), pl.ds(128, 96)],
                        bufb.at[slot], sem))
        return cps

    def _start(ck, slot, sem):
        for src, dst, s in _in_copies(ck, slot, sem):
            pltpu.async_copy(src, dst, s)

    def _wait(ck, slot, sem):
        for src, dst, s in _in_copies(ck, slot, sem):
            pltpu.make_async_copy(src, dst, s).wait()

    def _compute_chunk(ck, slot):
        ums = [ubuf[pl.ds(j * 16, 16)] for j in range(VREGS_PER_ROW)]

        def _row(r, carry2):
            row = ck * RPC + r
            vminus = row.astype(jnp.float32) - 112.0
            for g in range(VREGS_PER_ROW // GROUP):
                xys, wins, rvs, gvs, bvs = [], [], [], [], []
                for j in range(g * GROUP, (g + 1) * GROUP):
                    if j < 8:
                        sl = (slot, r, pl.ds(16 * j, 16))
                        dv = da[sl]
                    else:
                        sl = (slot, r, pl.ds(16 * j - 128, 16))
                        dv = db_[sl]
                    xf = ums[j] * dv / 112.0
                    yf = vminus * dv / 112.0
                    ix = ((xf + 1.0) * 32.0).astype(jnp.int32)
                    iy = ((yf + 1.0) * 32.0).astype(jnp.int32)
                    iz = ((dv + 1.0) * 32.0).astype(jnp.int32)
                    ok = (dv > 0.1) & (dv < ONE_MINUS) & (iy < 64)
                    ok &= (ix >> 3) == ss
                    lidx = ((ix & 7) << 11) | (iy << 5) | (iz & 31)
                    _, win = plsc.scan_count(lidx, mask=ok)
                    xys.append(lidx)
                    wins.append(win)
                    rvs.append(ra[sl] if j < 8 else rb[sl])
                    gvs.append(ga[sl] if j < 8 else gb[sl])
                    bvs.append(ba[sl] if j < 8 else bb_[sl])
                for t in range(GROUP):
                    plsc.store_scatter(slab_r, [xys[t]], rvs[t],
                                       mask=wins[t])
                    plsc.store_scatter(slab_g, [xys[t]], gvs[t],
                                       mask=wins[t])
                    plsc.store_scatter(slab_b, [xys[t]], bvs[t],
                                       mask=wins[t])
            return carry2

        lax.fori_loop(0, RPC, _row, 0)

    _start(0, 0, sem0)
    lax.fori_loop(0, SLAB // 16, _zero, 0, unroll=4)

    def _pair(k, carry):
        ck0 = 2 * k
        _start(ck0 + 1, 1, sem1)
        _wait(ck0, 0, sem0)
        _compute_chunk(ck0, 0)

        @pl.when(ck0 + 2 < NUM_CHUNKS)
        def _():
            _start(ck0 + 2, 0, sem0)

        _wait(ck0 + 1, 1, sem1)
        _compute_chunk(ck0 + 1, 1)
        return carry

    lax.fori_loop(0, NUM_CHUNKS // 2, _pair, 0)

    zeros_row = jnp.zeros((16,), jnp.float32)

    def _zstage(y, carry):
        for stg in (stg0, stg1, stg2, stg3):
            stg[y, pl.ds(0, 16)] = zeros_row
            stg[y, pl.ds(16, 16)] = zeros_row
        return carry

    lax.fori_loop(0, VOX, _zstage, 0, unroll=4)

    planes = [(c, xl) for c in range(3) for xl in range(8)]
    slabs = (slab_r, slab_g, slab_b)
    stgs = (stg0, stg1, stg2, stg3)
    sems = (sem0, sem1, sem2, sem3)
    NST = 4

    def _repack(slab, stg, xl):
        def _rrow(y, carry):
            base = (xl * VOX + y) * ZH
            stg[y, pl.ds(32, 16)] = slab[pl.ds(base, 16)]
            stg[y, pl.ds(48, 16)] = slab[pl.ds(base + 16, 16)]
            return carry

        lax.fori_loop(0, VOX, _rrow, 0, unroll=4)

    for i, (c, xl) in enumerate(planes):
        stg = stgs[i % NST]
        sem = sems[i % NST]
        if i >= NST:
            pc, pxl = planes[i - NST]
            pltpu.make_async_copy(
                stg,
                out_hbm.at[bb, pc, 8 * ss + pxl], sem).wait()
        _repack(slabs[c], stg, xl)
        pltpu.async_copy(stg,
                         out_hbm.at[bb, c, 8 * ss + xl], sem)
    for i in range(len(planes) - NST, len(planes)):
        c, xl = planes[i]
        pltpu.make_async_copy(stgs[i % NST],
                              out_hbm.at[bb, c, 8 * ss + xl],
                              sems[i % NST]).wait()


@jax.jit
def kernel(rgb, depth):
    um = jnp.asarray(np.arange(W, dtype=np.float32) - 112.0)
    mesh = plsc.VectorSubcoreMesh(core_axis_name="c", subcore_axis_name="s")
    call = functools.partial(
        pl.kernel,
        mesh=mesh,
        compiler_params=pltpu.CompilerParams(needs_layout_passes=False),
        out_type=jax.ShapeDtypeStruct((B, 3, VOX, VOX, VOX), jnp.float32),
        scratch_types=[
            pltpu.VMEM((SLAB,), jnp.float32),
            pltpu.VMEM((SLAB,), jnp.float32),
            pltpu.VMEM((SLAB,), jnp.float32),
            pltpu.VMEM((2, RPC, 128), jnp.float32),
            pltpu.VMEM((2, RPC, 96), jnp.float32),
            pltpu.VMEM((2, RPC, 128), jnp.float32),
            pltpu.VMEM((2, RPC, 96), jnp.float32),
            pltpu.VMEM((2, RPC, 128), jnp.float32),
            pltpu.VMEM((2, RPC, 96), jnp.float32),
            pltpu.VMEM((2, RPC, 128), jnp.float32),
            pltpu.VMEM((2, RPC, 96), jnp.float32),
            pltpu.VMEM((W,), jnp.float32),
            pltpu.VMEM((VOX, VOX), jnp.float32),
            pltpu.VMEM((VOX, VOX), jnp.float32),
            pltpu.VMEM((VOX, VOX), jnp.float32),
            pltpu.VMEM((VOX, VOX), jnp.float32),
            pltpu.SemaphoreType.DMA,
            pltpu.SemaphoreType.DMA,
            pltpu.SemaphoreType.DMA,
            pltpu.SemaphoreType.DMA,
        ],
    )(_voxel_body)
    return call(rgb, depth, um)


# speedup vs baseline: 1.3459x; 1.0538x over previous
import functools

import jax
import jax.numpy as jnp
import numpy as np
from jax import lax
from jax.experimental import pallas as pl
from jax.experimental.pallas import tpu as pltpu, tpu_sc as plsc

B = 4
H = 224
W = 224
VOX = 64
RPC = 16
NUM_CHUNKS = H // RPC
VREGS_PER_ROW = W // 16
GROUP = 7
ZH = 32
SLAB = 8 * VOX * ZH
ONE_MINUS = float(1.0 - 2.0 ** -24)


def _voxel_body(rgb_hbm, depth_hbm, um_hbm, out_hbm,
                slab_r, slab_g, slab_b,
                da, db_, ra, rb, ga, gb, ba, bb_, ubuf,
                stg0, stg1, stg2, stg3,
                sem0, sem1, sem2, sem3):
    wid = lax.axis_index("c") * 16 + lax.axis_index("s")
    bb = wid // 8
    ss = wid % 8

    pltpu.sync_copy(um_hbm, ubuf)

    zeros16 = jnp.zeros((16,), jnp.float32)

    def _zero(i, carry):
        slab_r[pl.ds(i * 16, 16)] = zeros16
        slab_g[pl.ds(i * 16, 16)] = zeros16
        slab_b[pl.ds(i * 16, 16)] = zeros16
        return carry

    def _in_copies(ck, slot, sem):
        r0 = ck * RPC
        cps = []
        for plane, bufa, bufb in (
            (depth_hbm.at[bb, 0], da, db_),
            (rgb_hbm.at[bb, 0], ra, rb),
            (rgb_hbm.at[bb, 1], ga, gb),
            (rgb_hbm.at[bb, 2], ba, bb_),
        ):
            cps.append((plane.at[pl.ds(r0, RPC), pl.ds(0, 128)],
                        bufa.at[slot], sem))
            cps.append((plane.at[pl.ds(r0, RPC), pl.ds(128, 96)],
                        bufb.at[slot], sem))
        return cps

    def _start(ck, slot, sem):
        for src, dst, s in _in_copies(ck, slot, sem):
            pltpu.async_copy(src, dst, s)

    def _wait(ck, slot, sem):
        for src, dst, s in _in_copies(ck, slot, sem):
            pltpu.make_async_copy(src, dst, s).wait()

    def _compute_chunk(ck, slot):
        ums = [ubuf[pl.ds(j * 16, 16)] for j in range(VREGS_PER_ROW)]

        def _row(r, carry2):
            row = ck * RPC + r
            vminus = row.astype(jnp.float32) - 112.0
            for g in range(VREGS_PER_ROW // GROUP):
                xys, wins, rvs, gvs, bvs = [], [], [], [], []
                for j in range(g * GROUP, (g + 1) * GROUP):
                    if j < 8:
                        sl = (slot, r, pl.ds(16 * j, 16))
                        dv = da[sl]
                    else:
                        sl = (slot, r, pl.ds(16 * j - 128, 16))
                        dv = db_[sl]
                    xf = ums[j] * dv / 112.0
                    yf = vminus * dv / 112.0
                    ix = ((xf + 1.0) * 32.0).astype(jnp.int32)
                    iy = ((yf + 1.0) * 32.0).astype(jnp.int32)
                    iz = ((dv + 1.0) * 32.0).astype(jnp.int32)
                    ok = (dv > 0.1) & (dv < ONE_MINUS) & (iy < 64)
                    ok &= (ix >> 3) == ss
                    lidx = ((ix & 7) << 11) | (iy << 5) | (iz & 31)
                    win = ok
                    xys.append(lidx)
                    wins.append(win)
                    rvs.append(ra[sl] if j < 8 else rb[sl])
                    gvs.append(ga[sl] if j < 8 else gb[sl])
                    bvs.append(ba[sl] if j < 8 else bb_[sl])
                for t in range(GROUP):
                    plsc.store_scatter(slab_r, [xys[t]], rvs[t],
                                       mask=wins[t])
                    plsc.store_scatter(slab_g, [xys[t]], gvs[t],
                                       mask=wins[t])
                    plsc.store_scatter(slab_b, [xys[t]], bvs[t],
                                       mask=wins[t])
            return carry2

        lax.fori_loop(0, RPC, _row, 0)

    _start(0, 0, sem0)
    lax.fori_loop(0, SLAB // 16, _zero, 0, unroll=4)

    def _pair(k, carry):
        ck0 = 2 * k
        _start(ck0 + 1, 1, sem1)
        _wait(ck0, 0, sem0)
        _compute_chunk(ck0, 0)

        @pl.when(ck0 + 2 < NUM_CHUNKS)
        def _():
            _start(ck0 + 2, 0, sem0)

        _wait(ck0 + 1, 1, sem1)
        _compute_chunk(ck0 + 1, 1)
        return carry

    lax.fori_loop(0, NUM_CHUNKS // 2, _pair, 0)

    zeros_row = jnp.zeros((16,), jnp.float32)

    def _zstage(y, carry):
        for stg in (stg0, stg1, stg2, stg3):
            stg[y, pl.ds(0, 16)] = zeros_row
            stg[y, pl.ds(16, 16)] = zeros_row
        return carry

    lax.fori_loop(0, VOX, _zstage, 0, unroll=4)

    planes = [(c, xl) for c in range(3) for xl in range(8)]
    slabs = (slab_r, slab_g, slab_b)
    stgs = (stg0, stg1, stg2, stg3)
    sems = (sem0, sem1, sem2, sem3)
    NST = 4

    def _repack(slab, stg, xl):
        def _rrow(y, carry):
            base = (xl * VOX + y) * ZH
            stg[y, pl.ds(32, 16)] = slab[pl.ds(base, 16)]
            stg[y, pl.ds(48, 16)] = slab[pl.ds(base + 16, 16)]
            return carry

        lax.fori_loop(0, VOX, _rrow, 0, unroll=4)

    for i, (c, xl) in enumerate(planes):
        stg = stgs[i % NST]
        sem = sems[i % NST]
        if i >= NST:
            pc, pxl = planes[i - NST]
            pltpu.make_async_copy(
                stg,
                out_hbm.at[bb, pc, 8 * ss + pxl], sem).wait()
        _repack(slabs[c], stg, xl)
        pltpu.async_copy(stg,
                         out_hbm.at[bb, c, 8 * ss + xl], sem)
    for i in range(len(planes) - NST, len(planes)):
        c, xl = planes[i]
        pltpu.make_async_copy(stgs[i % NST],
                              out_hbm.at[bb, c, 8 * ss + xl],
                              sems[i % NST]).wait()


@jax.jit
def kernel(rgb, depth):
    um = jnp.asarray(np.arange(W, dtype=np.float32) - 112.0)
    mesh = plsc.VectorSubcoreMesh(core_axis_name="c", subcore_axis_name="s")
    call = functools.partial(
        pl.kernel,
        mesh=mesh,
        compiler_params=pltpu.CompilerParams(needs_layout_passes=False),
        out_type=jax.ShapeDtypeStruct((B, 3, VOX, VOX, VOX), jnp.float32),
        scratch_types=[
            pltpu.VMEM((SLAB,), jnp.float32),
            pltpu.VMEM((SLAB,), jnp.float32),
            pltpu.VMEM((SLAB,), jnp.float32),
            pltpu.VMEM((2, RPC, 128), jnp.float32),
            pltpu.VMEM((2, RPC, 96), jnp.float32),
            pltpu.VMEM((2, RPC, 128), jnp.float32),
            pltpu.VMEM((2, RPC, 96), jnp.float32),
            pltpu.VMEM((2, RPC, 128), jnp.float32),
            pltpu.VMEM((2, RPC, 96), jnp.float32),
            pltpu.VMEM((2, RPC, 128), jnp.float32),
            pltpu.VMEM((2, RPC, 96), jnp.float32),
            pltpu.VMEM((W,), jnp.float32),
            pltpu.VMEM((VOX, VOX), jnp.float32),
            pltpu.VMEM((VOX, VOX), jnp.float32),
            pltpu.VMEM((VOX, VOX), jnp.float32),
            pltpu.VMEM((VOX, VOX), jnp.float32),
            pltpu.SemaphoreType.DMA,
            pltpu.SemaphoreType.DMA,
            pltpu.SemaphoreType.DMA,
            pltpu.SemaphoreType.DMA,
        ],
    )(_voxel_body)
    return call(rgb, depth, um)
